# baseline (speedup 1.0000x reference)
import functools

import jax
import jax.numpy as jnp
from jax import lax
from jax.experimental import pallas as pl
from jax.experimental.pallas import tpu as pltpu
from jax.experimental.pallas import tpu_sc as plsc

N = 2048
D = 768
FF = 3072
E = 8
K = 2
TOTAL = N * K

TM = 256
NT = TOTAL // TM
S = NT + E - 1
NF = 2
FFC = FF // NF

BT = 256


def _router_body(x_ref, rw_ref, p_ref, w_ref, counts_ref, ts_ref):
    x = x_ref[...]
    rw = rw_ref[...]
    logits = lax.dot_general(x, rw, (((1,), (1,)), ((), ())),
                             preferred_element_type=jnp.float32)
    m = jnp.max(logits, axis=-1, keepdims=True)
    ex = jnp.exp(logits - m)
    probs = ex / jnp.sum(ex, axis=-1, keepdims=True)

    colid = lax.broadcasted_iota(jnp.int32, (N, E), 1)
    m0 = jnp.max(probs, axis=-1, keepdims=True)
    a0 = jnp.min(jnp.where(probs == m0, colid, E), axis=-1, keepdims=True)
    probs1 = jnp.where(colid == a0, -jnp.inf, probs)
    m1 = jnp.max(probs1, axis=-1, keepdims=True)
    a1 = jnp.min(jnp.where(probs1 == m1, colid, E), axis=-1, keepdims=True)

    oh0 = (colid == a0).astype(jnp.float32)
    oh1 = (colid == a1).astype(jnp.float32)
    oh = jnp.concatenate([oh0, oh1], axis=0)

    tri = (lax.broadcasted_iota(jnp.int32, (BT, BT), 0)
           > lax.broadcasted_iota(jnp.int32, (BT, BT), 1)).astype(jnp.float32)
    blocks = []
    carry = jnp.zeros((1, E), jnp.float32)
    for b in range(TOTAL // BT):
        blk = oh[b * BT:(b + 1) * BT]
        pre = lax.dot_general(tri, blk, (((1,), (0,)), ((), ())),
                              precision=lax.Precision.HIGHEST,
                              preferred_element_type=jnp.float32) + carry
        blocks.append(pre)
        carry = carry + jnp.sum(blk, axis=0, keepdims=True)
    cnt = jnp.concatenate(blocks, axis=0)
    total = carry

    zeros1 = jnp.zeros((1, 1), jnp.float32)
    cols = [zeros1]
    run = zeros1
    for e in range(E - 1):
        run = run + total[:, e:e + 1]
        cols.append(run)
    offs = jnp.concatenate(cols, axis=1)

    rank = jnp.sum(oh * cnt, axis=1, keepdims=True)
    start = jnp.sum(oh * offs, axis=1, keepdims=True)
    pcol = rank + start
    p_ref[...] = pcol.astype(jnp.int32)
    w_ref[...] = jnp.concatenate([m0, m1], axis=0)
    counts_ref[...] = total.astype(jnp.int32)

    tv = jnp.remainder(
        lax.broadcasted_iota(jnp.int32, (1, TOTAL), 1), N).astype(jnp.float32)
    ts_blocks = []
    for jb in range(TOTAL // BT):
        j_ids = (jb * BT + lax.broadcasted_iota(jnp.int32, (1, BT), 1)
                 ).astype(jnp.float32)
        cmpm = (pcol == j_ids).astype(jnp.float32)
        ts_blocks.append(
            lax.dot_general(tv, cmpm, (((1,), (0,)), ((), ())),
                            precision=lax.Precision.HIGHEST,
                            preferred_element_type=jnp.float32))
    ts_ref[...] = jnp.concatenate(ts_blocks, axis=1).astype(jnp.int32)


def _router(x2, router_w):
    return pl.pallas_call(
        _router_body,
        out_shape=(
            jax.ShapeDtypeStruct((TOTAL, 1), jnp.int32),
            jax.ShapeDtypeStruct((TOTAL, 1), jnp.float32),
            jax.ShapeDtypeStruct((1, E), jnp.int32),
            jax.ShapeDtypeStruct((1, TOTAL), jnp.int32),
        ),
    )(x2, router_w)


SQ = S * NF


def _make_meta(counts):
    counts = counts.astype(jnp.int32)
    ends = jnp.cumsum(counts)
    starts = ends - counts
    t_first = starts // TM
    t_last = jnp.where(counts > 0, (ends - 1) // TM, t_first - 1)
    steps_e = jnp.maximum(t_last - t_first + 1, 0)
    flat_e = steps_e * NF
    cum_flat = jnp.cumsum(flat_e)
    cum_flat_excl = cum_flat - flat_e
    q = jnp.arange(SQ, dtype=jnp.int32)
    nbefore = jnp.sum((q[:, None] >= cum_flat[None, :]).astype(jnp.int32),
                      axis=1)
    pad = q >= cum_flat[E - 1]
    e_q = jnp.minimum(nbefore, E - 1)
    r = q - cum_flat_excl[e_q]
    se = jnp.maximum(steps_e[e_q], 1)
    f_q = r // se
    t_q = t_first[e_q] + r % se
    e_q = jnp.where(pad, E - 1, e_q)
    f_q = jnp.where(pad, NF - 1, f_q)
    t_q = jnp.where(pad, NT - 1, t_q)
    lo = jnp.where(pad, 0, jnp.maximum(starts[e_q], t_q * TM))
    hi = jnp.where(pad, 0, jnp.minimum(ends[e_q], (t_q + 1) * TM))
    earlier = (t_q[None, :] == t_q[:, None]) & (q[None, :] < q[:, None])
    init = (jnp.sum(earlier.astype(jnp.int32), axis=1) == 0) & ~pad
    return (t_q.astype(jnp.int32), e_q.astype(jnp.int32),
            f_q.astype(jnp.int32), lo.astype(jnp.int32),
            hi.astype(jnp.int32), init.astype(jnp.int32))


def _mlp_body(tm_ref, em_ref, fm_ref, lo_ref, hi_ref, in_ref,
              xs_ref, w1_ref, w2_ref, out_ref):
    q = pl.program_id(0)
    t = tm_ref[q]
    lo = lo_ref[q]
    hi = hi_ref[q]

    rows = t * TM + lax.broadcasted_iota(jnp.int32, (TM, 1), 0)
    mask = (rows >= lo) & (rows < hi)
    xm = jnp.where(mask, xs_ref[...], 0.0)

    h = lax.dot_general(xm, w1_ref[...], (((1,), (1,)), ((), ())),
                        preferred_element_type=jnp.float32)
    h = 0.5 * h * (1.0 + lax.erf(h * 0.7071067811865476))
    y = lax.dot_general(h, w2_ref[...], (((1,), (1,)), ((), ())),
                        preferred_element_type=jnp.float32)

    sl = pl.ds(t * TM, TM)

    @pl.when(in_ref[q] == 1)
    def _():
        out_ref[sl, :] = y

    @pl.when(in_ref[q] == 0)
    def _():
        out_ref[sl, :] += y


def _mlp(meta, xs, w1, w2):
    t_q, e_q, f_q, lo, hi, init = meta
    grid_spec = pltpu.PrefetchScalarGridSpec(
        num_scalar_prefetch=6,
        grid=(SQ,),
        in_specs=[
            pl.BlockSpec((TM, D),
                         lambda q, tm, em, fm, lo, hi, ii: (tm[q], 0)),
            pl.BlockSpec((None, FFC, D),
                         lambda q, tm, em, fm, lo, hi, ii: (em[q], fm[q], 0)),
            pl.BlockSpec((None, D, FFC),
                         lambda q, tm, em, fm, lo, hi, ii: (em[q], 0, fm[q])),
        ],
        out_specs=pl.BlockSpec((TOTAL, D),
                               lambda q, tm, em, fm, lo, hi, ii: (0, 0)),
    )
    return pl.pallas_call(
        _mlp_body,
        grid_spec=grid_spec,
        out_shape=jax.ShapeDtypeStruct((TOTAL, D), jnp.float32),
    )(t_q, e_q, f_q, lo, hi, init, xs, w1, w2)


NW = 32
CH = TOTAL // NW
CHD = CH // 2


@functools.cache
def _get_dispatch():
    mesh = plsc.VectorSubcoreMesh(core_axis_name="c", subcore_axis_name="s")

    @functools.partial(
        pl.kernel,
        mesh=mesh,
        out_type=jax.ShapeDtypeStruct((TOTAL, D), jnp.float32),
        scratch_types=[
            pltpu.VMEM((CHD,), jnp.int32),
            pltpu.VMEM((CHD, D), jnp.float32),
            pltpu.SemaphoreType.DMA,
        ],
    )
    def _dispatch(x_hbm, ts_hbm, xs_hbm, idx_v, rows_v, sem):
        wid = lax.axis_index("s") * 2 + lax.axis_index("c")
        for c in range(CH // CHD):
            base = wid * CH + c * CHD
            pltpu.sync_copy(ts_hbm.at[pl.ds(base, CHD)], idx_v)
            pltpu.async_copy(x_hbm.at[idx_v], rows_v, sem).wait()
            pltpu.sync_copy(rows_v, xs_hbm.at[pl.ds(base, CHD)])

    return _dispatch


CT = N // NW


@functools.cache
def _get_gather2():
    mesh = plsc.VectorSubcoreMesh(core_axis_name="c", subcore_axis_name="s")

    @functools.partial(
        pl.kernel,
        mesh=mesh,
        out_type=(
            jax.ShapeDtypeStruct((N, D), jnp.float32),
            jax.ShapeDtypeStruct((N, D), jnp.float32),
        ),
        scratch_types=[
            pltpu.VMEM((CT,), jnp.int32),
            pltpu.VMEM((CT,), jnp.int32),
            pltpu.VMEM((CT, D), jnp.float32),
            pltpu.VMEM((CT, D), jnp.float32),
            pltpu.SemaphoreType.DMA,
        ],
    )
    def _gather2(ys_hbm, p_hbm, g0_hbm, g1_hbm, i0_v, i1_v, a_v, b_v, sem):
        wid = lax.axis_index("s") * 2 + lax.axis_index("c")
        base = wid * CT
        pltpu.sync_copy(p_hbm.at[pl.ds(base, CT)], i0_v)
        pltpu.sync_copy(p_hbm.at[pl.ds(N + base, CT)], i1_v)
        pltpu.async_copy(ys_hbm.at[i0_v], a_v, sem).wait()
        pltpu.async_copy(ys_hbm.at[i1_v], b_v, sem).wait()
        pltpu.sync_copy(a_v, g0_hbm.at[pl.ds(base, CT)])
        pltpu.sync_copy(b_v, g1_hbm.at[pl.ds(base, CT)])

    return _gather2


def _blend_body(g0_ref, g1_ref, w0_ref, w1_ref, out_ref):
    out_ref[...] = w0_ref[...] * g0_ref[...] + w1_ref[...] * g1_ref[...]


def _blend(g0, g1, w0, w1):
    nb = 2
    bs = N // nb
    return pl.pallas_call(
        _blend_body,
        grid=(nb,),
        in_specs=[
            pl.BlockSpec((bs, D), lambda i: (i, 0)),
            pl.BlockSpec((bs, D), lambda i: (i, 0)),
            pl.BlockSpec((bs, 1), lambda i: (i, 0)),
            pl.BlockSpec((bs, 1), lambda i: (i, 0)),
        ],
        out_specs=pl.BlockSpec((bs, D), lambda i: (i, 0)),
        out_shape=jax.ShapeDtypeStruct((N, D), jnp.float32),
    )(g0, g1, w0, w1)


def kernel(x, router_w, w1, w2):
    b, t, d = x.shape
    x2 = x.reshape(t, d)
    p2, wg2, counts2, ts2 = _router(x2, router_w)
    p = p2.reshape(-1)
    ts = ts2.reshape(-1)
    counts = counts2.reshape(-1)
    meta = _make_meta(counts)
    xs = _get_dispatch()(x2, ts)
    ys = _mlp(meta, xs, w1, w2)
    g0, g1 = _get_gather2()(ys, p)
    out = _blend(g0, g1, wg2[:N], wg2[N:])
    return out.reshape(b, t, d)


# speedup vs baseline: 1.2469x; 1.2469x over previous
import functools

import jax
import jax.numpy as jnp
from jax import lax
from jax.experimental import pallas as pl
from jax.experimental.pallas import tpu as pltpu
from jax.experimental.pallas import tpu_sc as plsc

N = 2048
D = 768
FF = 3072
E = 8
K = 2
TOTAL = N * K

TM = 512
NT = TOTAL // TM
S = NT + E - 1
NF = 1
FFC = FF // NF

BT = 256


def _router_body(x_ref, rw_ref, p_ref, w_ref, counts_ref, ts_ref):
    x = x_ref[...]
    rw = rw_ref[...]
    logits = lax.dot_general(x, rw, (((1,), (1,)), ((), ())),
                             preferred_element_type=jnp.float32)
    m = jnp.max(logits, axis=-1, keepdims=True)
    ex = jnp.exp(logits - m)
    probs = ex / jnp.sum(ex, axis=-1, keepdims=True)

    colid = lax.broadcasted_iota(jnp.int32, (N, E), 1)
    m0 = jnp.max(probs, axis=-1, keepdims=True)
    a0 = jnp.min(jnp.where(probs == m0, colid, E), axis=-1, keepdims=True)
    probs1 = jnp.where(colid == a0, -jnp.inf, probs)
    m1 = jnp.max(probs1, axis=-1, keepdims=True)
    a1 = jnp.min(jnp.where(probs1 == m1, colid, E), axis=-1, keepdims=True)

    oh0 = (colid == a0).astype(jnp.float32)
    oh1 = (colid == a1).astype(jnp.float32)
    oh = jnp.concatenate([oh0, oh1], axis=0)

    tri = (lax.broadcasted_iota(jnp.int32, (BT, BT), 0)
           > lax.broadcasted_iota(jnp.int32, (BT, BT), 1)).astype(jnp.float32)
    blocks = []
    carry = jnp.zeros((1, E), jnp.float32)
    for b in range(TOTAL // BT):
        blk = oh[b * BT:(b + 1) * BT]
        pre = lax.dot_general(tri, blk, (((1,), (0,)), ((), ())),
                              precision=lax.Precision.HIGHEST,
                              preferred_element_type=jnp.float32) + carry
        blocks.append(pre)
        carry = carry + jnp.sum(blk, axis=0, keepdims=True)
    cnt = jnp.concatenate(blocks, axis=0)
    total = carry

    zeros1 = jnp.zeros((1, 1), jnp.float32)
    cols = [zeros1]
    run = zeros1
    for e in range(E - 1):
        run = run + total[:, e:e + 1]
        cols.append(run)
    offs = jnp.concatenate(cols, axis=1)

    rank = jnp.sum(oh * cnt, axis=1, keepdims=True)
    start = jnp.sum(oh * offs, axis=1, keepdims=True)
    pcol = rank + start
    p_ref[...] = pcol.astype(jnp.int32)
    w_ref[...] = jnp.concatenate([m0, m1], axis=0)
    counts_ref[...] = total.astype(jnp.int32)

    tv = jnp.remainder(
        lax.broadcasted_iota(jnp.int32, (1, TOTAL), 1), N).astype(jnp.float32)
    ts_blocks = []
    for jb in range(TOTAL // BT):
        j_ids = (jb * BT + lax.broadcasted_iota(jnp.int32, (1, BT), 1)
                 ).astype(jnp.float32)
        cmpm = (pcol == j_ids).astype(jnp.float32)
        ts_blocks.append(
            lax.dot_general(tv, cmpm, (((1,), (0,)), ((), ())),
                            precision=lax.Precision.HIGHEST,
                            preferred_element_type=jnp.float32))
    ts_ref[...] = jnp.concatenate(ts_blocks, axis=1).astype(jnp.int32)


def _router(x2, router_w):
    return pl.pallas_call(
        _router_body,
        out_shape=(
            jax.ShapeDtypeStruct((TOTAL, 1), jnp.int32),
            jax.ShapeDtypeStruct((TOTAL, 1), jnp.float32),
            jax.ShapeDtypeStruct((1, E), jnp.int32),
            jax.ShapeDtypeStruct((1, TOTAL), jnp.int32),
        ),
    )(x2, router_w)


SQ = S * NF


def _make_meta(counts):
    counts = counts.astype(jnp.int32)
    ends = jnp.cumsum(counts)
    starts = ends - counts
    t_first = starts // TM
    t_last = jnp.where(counts > 0, (ends - 1) // TM, t_first - 1)
    steps_e = jnp.maximum(t_last - t_first + 1, 0)
    flat_e = steps_e * NF
    cum_flat = jnp.cumsum(flat_e)
    cum_flat_excl = cum_flat - flat_e
    q = jnp.arange(SQ, dtype=jnp.int32)
    nbefore = jnp.sum((q[:, None] >= cum_flat[None, :]).astype(jnp.int32),
                      axis=1)
    pad = q >= cum_flat[E - 1]
    e_q = jnp.minimum(nbefore, E - 1)
    r = q - cum_flat_excl[e_q]
    se = jnp.maximum(steps_e[e_q], 1)
    f_q = r // se
    t_q = t_first[e_q] + r % se
    e_q = jnp.where(pad, E - 1, e_q)
    f_q = jnp.where(pad, NF - 1, f_q)
    t_q = jnp.where(pad, NT - 1, t_q)
    lo = jnp.where(pad, 0, jnp.maximum(starts[e_q], t_q * TM))
    hi = jnp.where(pad, 0, jnp.minimum(ends[e_q], (t_q + 1) * TM))
    earlier = (t_q[None, :] == t_q[:, None]) & (q[None, :] < q[:, None])
    init = (jnp.sum(earlier.astype(jnp.int32), axis=1) == 0) & ~pad
    return (t_q.astype(jnp.int32), e_q.astype(jnp.int32),
            f_q.astype(jnp.int32), lo.astype(jnp.int32),
            hi.astype(jnp.int32), init.astype(jnp.int32))


def _mlp_body(tm_ref, em_ref, fm_ref, lo_ref, hi_ref, in_ref,
              xs_ref, w1_ref, w2_ref, out_ref):
    q = pl.program_id(0)
    t = tm_ref[q]
    lo = lo_ref[q]
    hi = hi_ref[q]

    rows = t * TM + lax.broadcasted_iota(jnp.int32, (TM, 1), 0)
    mask = (rows >= lo) & (rows < hi)
    xm = jnp.where(mask, xs_ref[...], 0.0)

    h = lax.dot_general(xm, w1_ref[...], (((1,), (1,)), ((), ())),
                        preferred_element_type=jnp.float32)
    h = 0.5 * h * (1.0 + lax.erf(h * 0.7071067811865476))
    y = lax.dot_general(h, w2_ref[...], (((1,), (1,)), ((), ())),
                        preferred_element_type=jnp.float32)

    sl = pl.ds(t * TM, TM)

    @pl.when(in_ref[q] == 1)
    def _():
        out_ref[sl, :] = y

    @pl.when(in_ref[q] == 0)
    def _():
        out_ref[sl, :] += y


def _mlp(meta, xs, w1, w2):
    t_q, e_q, f_q, lo, hi, init = meta
    grid_spec = pltpu.PrefetchScalarGridSpec(
        num_scalar_prefetch=6,
        grid=(SQ,),
        in_specs=[
            pl.BlockSpec((TM, D),
                         lambda q, tm, em, fm, lo, hi, ii: (tm[q], 0)),
            pl.BlockSpec((None, FFC, D),
                         lambda q, tm, em, fm, lo, hi, ii: (em[q], fm[q], 0)),
            pl.BlockSpec((None, D, FFC),
                         lambda q, tm, em, fm, lo, hi, ii: (em[q], 0, fm[q])),
        ],
        out_specs=pl.BlockSpec((TOTAL, D),
                               lambda q, tm, em, fm, lo, hi, ii: (0, 0)),
    )
    return pl.pallas_call(
        _mlp_body,
        grid_spec=grid_spec,
        out_shape=jax.ShapeDtypeStruct((TOTAL, D), jnp.float32),
    )(t_q, e_q, f_q, lo, hi, init, xs, w1, w2)


NW = 32
CH = TOTAL // NW
CHD = CH // 2


@functools.cache
def _get_dispatch():
    mesh = plsc.VectorSubcoreMesh(core_axis_name="c", subcore_axis_name="s")

    @functools.partial(
        pl.kernel,
        mesh=mesh,
        out_type=jax.ShapeDtypeStruct((TOTAL, D), jnp.float32),
        scratch_types=[
            pltpu.VMEM((CHD,), jnp.int32),
            pltpu.VMEM((CHD, D), jnp.float32),
            pltpu.SemaphoreType.DMA,
        ],
    )
    def _dispatch(x_hbm, ts_hbm, xs_hbm, idx_v, rows_v, sem):
        wid = lax.axis_index("s") * 2 + lax.axis_index("c")
        for c in range(CH // CHD):
            base = wid * CH + c * CHD
            pltpu.sync_copy(ts_hbm.at[pl.ds(base, CHD)], idx_v)
            pltpu.async_copy(x_hbm.at[idx_v], rows_v, sem).wait()
            pltpu.sync_copy(rows_v, xs_hbm.at[pl.ds(base, CHD)])

    return _dispatch


CT = N // NW


@functools.cache
def _get_gather2():
    mesh = plsc.VectorSubcoreMesh(core_axis_name="c", subcore_axis_name="s")

    @functools.partial(
        pl.kernel,
        mesh=mesh,
        out_type=(
            jax.ShapeDtypeStruct((N, D), jnp.float32),
            jax.ShapeDtypeStruct((N, D), jnp.float32),
        ),
        scratch_types=[
            pltpu.VMEM((CT,), jnp.int32),
            pltpu.VMEM((CT,), jnp.int32),
            pltpu.VMEM((CT, D), jnp.float32),
            pltpu.VMEM((CT, D), jnp.float32),
            pltpu.SemaphoreType.DMA,
        ],
    )
    def _gather2(ys_hbm, p_hbm, g0_hbm, g1_hbm, i0_v, i1_v, a_v, b_v, sem):
        wid = lax.axis_index("s") * 2 + lax.axis_index("c")
        base = wid * CT
        pltpu.sync_copy(p_hbm.at[pl.ds(base, CT)], i0_v)
        pltpu.sync_copy(p_hbm.at[pl.ds(N + base, CT)], i1_v)
        pltpu.async_copy(ys_hbm.at[i0_v], a_v, sem).wait()
        pltpu.async_copy(ys_hbm.at[i1_v], b_v, sem).wait()
        pltpu.sync_copy(a_v, g0_hbm.at[pl.ds(base, CT)])
        pltpu.sync_copy(b_v, g1_hbm.at[pl.ds(base, CT)])

    return _gather2


def _blend_body(g0_ref, g1_ref, w0_ref, w1_ref, out_ref):
    out_ref[...] = w0_ref[...] * g0_ref[...] + w1_ref[...] * g1_ref[...]


def _blend(g0, g1, w0, w1):
    nb = 2
    bs = N // nb
    return pl.pallas_call(
        _blend_body,
        grid=(nb,),
        in_specs=[
            pl.BlockSpec((bs, D), lambda i: (i, 0)),
            pl.BlockSpec((bs, D), lambda i: (i, 0)),
            pl.BlockSpec((bs, 1), lambda i: (i, 0)),
            pl.BlockSpec((bs, 1), lambda i: (i, 0)),
        ],
        out_specs=pl.BlockSpec((bs, D), lambda i: (i, 0)),
        out_shape=jax.ShapeDtypeStruct((N, D), jnp.float32),
    )(g0, g1, w0, w1)


def kernel(x, router_w, w1, w2):
    b, t, d = x.shape
    x2 = x.reshape(t, d)
    p2, wg2, counts2, ts2 = _router(x2, router_w)
    p = p2.reshape(-1)
    ts = ts2.reshape(-1)
    counts = counts2.reshape(-1)
    meta = _make_meta(counts)
    xs = _get_dispatch()(x2, ts)
    ys = _mlp(meta, xs, w1, w2)
    g0, g1 = _get_gather2()(ys, p)
    out = _blend(g0, g1, wg2[:N], wg2[N:])
    return out.reshape(b, t, d)
